# baseline (speedup 1.0000x reference)
import functools

import jax
import jax.numpy as jnp
from jax import lax
from jax.experimental import pallas as pl
from jax.experimental.pallas import tpu as pltpu
from jax.experimental.pallas import tpu_sc as plsc

MAX_LAYERS = 16
ANGULAR = 512
HD = 128
N_POS = 4 * 8192
IDX_ROWS = N_POS // 128
NW = 32
ROWS_PER_W = IDX_ROWS // NW


def _prep_body(c_ref, radial_ref, angular_ref, idx_ref, comb_ref):
    c = c_ref[...]
    n = jnp.clip(c[0, :, 0], 0, MAX_LAYERS - 1)
    m = jnp.clip(c[0, :, 1], 0, ANGULAR - 1)
    idx_ref[...] = (n * ANGULAR + m).reshape(8, 128)

    i = pl.program_id(0)
    r = radial_ref[i // 2, :]
    comb_ref[...] = angular_ref[...] + r[None, :]


def _prep(spiral_coords, radial, angular):
    return pl.pallas_call(
        _prep_body,
        grid=(32,),
        in_specs=[
            pl.BlockSpec((1, 1024, 3), lambda i: (i // 8, i % 8, 0)),
            pl.BlockSpec((MAX_LAYERS, HD), lambda i: (0, 0)),
            pl.BlockSpec((ANGULAR // 2, HD), lambda i: (i % 2, 0)),
        ],
        out_specs=[
            pl.BlockSpec((8, 128), lambda i: (i, 0)),
            pl.BlockSpec((ANGULAR // 2, HD), lambda i: (i, 0)),
        ],
        out_shape=[
            jax.ShapeDtypeStruct((IDX_ROWS + 16, 128), jnp.int32),
            jax.ShapeDtypeStruct((MAX_LAYERS * ANGULAR, HD), jnp.float32),
        ],
    )(spiral_coords, radial, angular)


def _make_sc_gather(row_start, rows_per_w):
    n_out = 32 * rows_per_w * 128

    def body(comb_hbm, idx_hbm, out_hbm, idx_v, rows0, rows1, g0, g1):
        c = lax.axis_index("c")
        s = lax.axis_index("s")
        wid = s * 2 + c
        base = row_start + wid * rows_per_w
        obase = wid * rows_per_w
        aligned = pl.multiple_of((base // 8) * 8, 8)
        local = base - aligned
        pltpu.sync_copy(idx_hbm.at[pl.ds(aligned, 16)], idx_v)

        bufs = (rows0, rows1)
        sems = (g0, g1)
        cps = [None] * rows_per_w
        cps[0] = pltpu.async_copy(comb_hbm.at[idx_v.at[local]], bufs[0], sems[0])
        for j in range(rows_per_w):
            if j + 1 < rows_per_w:
                cps[j + 1] = pltpu.async_copy(
                    comb_hbm.at[idx_v.at[local + (j + 1)]],
                    bufs[(j + 1) % 2],
                    sems[(j + 1) % 2],
                )
            cps[j].wait()
            pltpu.sync_copy(bufs[j % 2], out_hbm.at[pl.ds((obase + j) * 128, 128)])

    return functools.partial(
        pl.kernel,
        out_type=jax.ShapeDtypeStruct((n_out, HD), jnp.float32),
        mesh=plsc.VectorSubcoreMesh(core_axis_name="c", subcore_axis_name="s"),
        scratch_types=[
            pltpu.VMEM((16, 128), jnp.int32),
            pltpu.VMEM((128, HD), jnp.float32),
            pltpu.VMEM((128, HD), jnp.float32),
            pltpu.SemaphoreType.DMA,
            pltpu.SemaphoreType.DMA,
        ],
    )(body)


ROWS_A = 64
ROWS_B = IDX_ROWS - ROWS_A
_sc_gather_a = _make_sc_gather(0, ROWS_A // 32)
_sc_gather_b = _make_sc_gather(ROWS_A, ROWS_B // 32)


def _add_body(h_ref, e_ref, o_ref):
    e = e_ref[...]
    o_ref[...] = h_ref[...] + jnp.concatenate([e] * 16, axis=1)


def _add_body_inplace(prev_ref, h_ref, e_ref, o_ref):
    del prev_ref
    e = e_ref[...]
    o_ref[...] = h_ref[...] + jnp.concatenate([e] * 16, axis=1)


_BR = 1024


def _broadcast_add_a(hidden2, enc_a):
    return pl.pallas_call(
        _add_body,
        grid=(ROWS_A * 128 // _BR,),
        in_specs=[
            pl.BlockSpec((_BR, 16 * HD), lambda i: (i, 0)),
            pl.BlockSpec((_BR, HD), lambda i: (i, 0)),
        ],
        out_specs=pl.BlockSpec((_BR, 16 * HD), lambda i: (i, 0)),
        out_shape=jax.ShapeDtypeStruct((N_POS, 16 * HD), jnp.float32),
    )(hidden2, enc_a)


def _broadcast_add_b(partial_out, hidden2, enc_b):
    off = ROWS_A * 128 // _BR
    return pl.pallas_call(
        _add_body_inplace,
        grid=(ROWS_B * 128 // _BR,),
        in_specs=[
            pl.BlockSpec(memory_space=pltpu.MemorySpace.HBM),
            pl.BlockSpec((_BR, 16 * HD), lambda i: (i + off, 0)),
            pl.BlockSpec((_BR, HD), lambda i: (i, 0)),
        ],
        out_specs=pl.BlockSpec((_BR, 16 * HD), lambda i: (i + off, 0)),
        out_shape=jax.ShapeDtypeStruct((N_POS, 16 * HD), jnp.float32),
        input_output_aliases={0: 0},
    )(partial_out, hidden2, enc_b)


def kernel(hidden_states, spiral_coords, radial_freq, angular_freq):
    batch_size, seq_len, dim = hidden_states.shape
    fused_idx, comb = _prep(
        spiral_coords.astype(jnp.int32), radial_freq[0, 0], angular_freq[0, 0]
    )
    enc_a = _sc_gather_a(comb, fused_idx)
    enc_b = _sc_gather_b(comb, fused_idx)
    hidden2 = hidden_states.reshape(N_POS, 16 * HD)
    out_a = _broadcast_add_a(hidden2, enc_a)
    out2 = _broadcast_add_b(out_a, hidden2, enc_b)
    return out2.reshape(batch_size, seq_len, dim)
